# Initial kernel scaffold; baseline (speedup 1.0000x reference)
import jax
import jax.numpy as jnp
from jax.experimental import pallas as pl


def kernel(x, edge_index, W1, b1, W2, b2):
    raise NotImplementedError("write your pallas kernel here")


# speedup vs baseline: 4.8276x; 4.8276x over previous
import functools

import jax
import jax.numpy as jnp
from jax import lax
from jax.experimental import pallas as pl
from jax.experimental.pallas import tpu as pltpu
from jax.experimental.pallas import tpu_sc as plsc

D = 128
NC = 2
NS = 16
NW = NC * NS
CHUNK = 128
MM_BLOCK = 2000


def _seg_sum_partials(h, src_r, dst_r, init, n_nodes, n_pad, k_chunks):
    rows_init = n_pad // NS
    rows_out = (n_nodes // NS) // 8 * 8
    mesh = plsc.VectorSubcoreMesh(
        core_axis_name="c", subcore_axis_name="s",
        num_cores=NC, num_subcores=NS)

    @functools.partial(
        pl.kernel,
        out_type=jax.ShapeDtypeStruct((NC, n_nodes, D), jnp.float32),
        mesh=mesh,
        scratch_types=[
            pltpu.VMEM((k_chunks, CHUNK), jnp.int32),
            pltpu.VMEM((k_chunks, CHUNK), jnp.int32),
            pltpu.VMEM((CHUNK, D), jnp.float32),
            pltpu.VMEM_SHARED((n_pad, D), jnp.float32),
            pltpu.SemaphoreType.DMA,
        ],
    )
    def kern(h_hbm, src_hbm, dst_hbm, init_hbm, out_hbm,
             src_v, dst_v, rows_v, acc, sem):
        cid = lax.axis_index("c")
        sid = lax.axis_index("s")
        w = cid * NS + sid
        pltpu.sync_copy(src_hbm.at[w], src_v)
        pltpu.sync_copy(dst_hbm.at[w], dst_v)
        pltpu.sync_copy(init_hbm.at[cid],
                        acc.at[pl.ds(sid * rows_init, rows_init)])
        plsc.subcore_barrier()

        def body(j, carry):
            pltpu.async_copy(h_hbm.at[src_v.at[j]], rows_v, sem).wait()
            pltpu.sync_copy(rows_v, acc.at[dst_v.at[j]], add=True)
            return carry

        lax.fori_loop(0, k_chunks, body, 0)
        plsc.subcore_barrier()
        pltpu.sync_copy(acc.at[pl.ds(sid * rows_out, rows_out)],
                        out_hbm.at[cid, pl.ds(sid * rows_out, rows_out)])

        @pl.when(sid == NS - 1)
        def _():
            tail = n_nodes - NS * rows_out
            if tail:
                pltpu.sync_copy(
                    acc.at[pl.ds(NS * rows_out, tail)],
                    out_hbm.at[cid, pl.ds(NS * rows_out, tail)])

    return kern(h, src_r, dst_r, init)


def _mm(x, w):
    n = x.shape[0]

    def body(x_ref, w_ref, o_ref):
        o_ref[...] = jnp.dot(x_ref[...], w_ref[...],
                             preferred_element_type=jnp.float32)

    return pl.pallas_call(
        body,
        grid=(n // MM_BLOCK,),
        in_specs=[pl.BlockSpec((MM_BLOCK, D), lambda i: (i, 0)),
                  pl.BlockSpec((D, D), lambda i: (0, 0))],
        out_specs=pl.BlockSpec((MM_BLOCK, D), lambda i: (i, 0)),
        out_shape=jax.ShapeDtypeStruct((n, D), jnp.float32),
    )(x, w)


def _fused_relu_mm(p, w):
    n = p.shape[1]

    def body(p_ref, w_ref, o_ref):
        h = jnp.maximum(p_ref[0] + p_ref[1], 0.0)
        o_ref[...] = jnp.dot(h, w_ref[...], preferred_element_type=jnp.float32)

    return pl.pallas_call(
        body,
        grid=(n // MM_BLOCK,),
        in_specs=[pl.BlockSpec((NC, MM_BLOCK, D), lambda i: (0, i, 0)),
                  pl.BlockSpec((D, D), lambda i: (0, 0))],
        out_specs=pl.BlockSpec((MM_BLOCK, D), lambda i: (i, 0)),
        out_shape=jax.ShapeDtypeStruct((n, D), jnp.float32),
    )(p, w)


def _partial_add(q):
    n = q.shape[1]

    def body(q_ref, o_ref):
        o_ref[...] = q_ref[0] + q_ref[1]

    return pl.pallas_call(
        body,
        grid=(n // MM_BLOCK,),
        in_specs=[pl.BlockSpec((NC, MM_BLOCK, D), lambda i: (0, i, 0))],
        out_specs=pl.BlockSpec((MM_BLOCK, D), lambda i: (i, 0)),
        out_shape=jax.ShapeDtypeStruct((n, D), jnp.float32),
    )(q)


def kernel(x, edge_index, W1, b1, W2, b2):
    n_nodes = x.shape[0]
    e = edge_index.shape[1]
    k_chunks = -(-e // (NW * CHUNK))
    e_pad = NW * CHUNK * k_chunks
    src = edge_index[0]
    dst = edge_index[1]
    if e_pad != e:
        pad = e_pad - e
        src = jnp.concatenate([src, jnp.zeros((pad,), jnp.int32)])
        dst = jnp.concatenate([dst, jnp.full((pad,), n_nodes, jnp.int32)])
    src_r = src.reshape(NW, k_chunks, CHUNK)
    dst_r = dst.reshape(NW, k_chunks, CHUNK)

    n_pad = -(-(n_nodes + 1) // NS) * NS
    rows_init = n_pad // NS
    zero_init = jnp.zeros((rows_init, D), jnp.float32)
    init1 = jnp.stack([jnp.broadcast_to(b1, (rows_init, D)), zero_init])
    init2 = jnp.stack([jnp.broadcast_to(b2, (rows_init, D)), zero_init])

    h1 = _mm(x, W1)
    p = _seg_sum_partials(h1, src_r, dst_r, init1, n_nodes, n_pad, k_chunks)
    h2 = _fused_relu_mm(p, W2)
    q = _seg_sum_partials(h2, src_r, dst_r, init2, n_nodes, n_pad, k_chunks)
    return _partial_add(q)
